# baseline (speedup 1.0000x reference)
import functools

import jax
import jax.numpy as jnp
from jax import lax
from jax.experimental import pallas as pl
from jax.experimental.pallas import tpu as pltpu
from jax.experimental.pallas import tpu_sc as plsc

B, F_, V, D = 4096, 26, 1000, 128
H1, H2 = 1024, 1024
CROSS_DEPTH = 3
FD = F_ * D

_NC, _NS = 2, 16
_NW = _NC * _NS
_CH = 128


def _gather_body(nch, tab, idx, out, idx_v, rows0, rows1, sem0, sem1):
    wid = lax.axis_index("s") * _NC + lax.axis_index("c")
    base = wid * nch
    pltpu.sync_copy(idx.at[wid], idx_v)

    def _start(ci, buf, sem):
        pltpu.async_copy(tab.at[idx_v.at[ci]], buf, sem)

    def _drain(ci, buf, sem):
        pltpu.make_async_copy(tab.at[idx_v.at[ci]], buf, sem).wait()
        off = pl.multiple_of((base + ci) * _CH, _CH)
        pltpu.sync_copy(buf, out.at[pl.ds(off, _CH)])

    bufs = ((rows0, sem0), (rows1, sem1))
    _start(0, rows0, sem0)
    if nch > 1:
        _start(1, rows1, sem1)
    for ci in range(nch):
        _drain(ci, *bufs[ci % 2])
        if ci + 2 < nch:
            _start(ci + 2, *bufs[ci % 2])


def _sc_gather(tables_flat, gidx3d):
    nw, nch, ch = gidx3d.shape
    mesh = plsc.VectorSubcoreMesh(core_axis_name="c", subcore_axis_name="s")
    k = functools.partial(
        pl.kernel,
        mesh=mesh,
        out_type=jax.ShapeDtypeStruct((nw * nch * ch, D), jnp.float32),
        scratch_types=[
            pltpu.VMEM((nch, ch), jnp.int32),
            pltpu.VMEM((ch, D), jnp.float32),
            pltpu.VMEM((ch, D), jnp.float32),
            pltpu.SemaphoreType.DMA,
            pltpu.SemaphoreType.DMA,
        ],
    )(functools.partial(_gather_body, nch))
    return k(tables_flat, gidx3d)


def _tc_body(emb_ref, xv_ref, e_ref, p_ref, uc_ref, w1_ref, b1_ref,
             w2_ref, b2_ref, wlb_ref, bl_ref, out_ref):
    xv = xv_ref[...]
    xv_hi = xv.astype(jnp.bfloat16)
    xv_lo = (xv - xv_hi.astype(jnp.float32)).astype(jnp.bfloat16)
    e = e_ref[...]
    xv_wide = (jnp.dot(xv_hi, e, preferred_element_type=jnp.float32)
               + jnp.dot(xv_lo, e, preferred_element_type=jnp.float32))
    x0 = emb_ref[...] * xv_wide
    a = jnp.sum(x0 * x0, axis=1, keepdims=True)
    t = jnp.dot(x0, p_ref[...], preferred_element_type=jnp.float32)
    p0 = t[:, 0:1]
    p1 = t[:, 1:2]
    q0 = t[:, 2:3]
    q1 = t[:, 3:4]
    r0 = t[:, 4:5]
    s0 = a
    s1 = s0 * p0 + q0 + a
    s2 = s1 * p1 + q1 + s1
    uc = uc_ref[...]
    cross = (r0 + s0 * uc[:, 0:1] + s1 * uc[:, 1:2] + s2 * uc[:, 2:3]
             + uc[:, 3:4])
    h = jnp.dot(x0.astype(jnp.bfloat16), w1_ref[...],
                preferred_element_type=jnp.float32)
    h = jnp.maximum(h + b1_ref[...], 0.0)
    h = jnp.dot(h.astype(jnp.bfloat16), w2_ref[...],
                preferred_element_type=jnp.float32)
    h = jnp.maximum(h + b2_ref[...], 0.0)
    out_ref[...] = (cross
                    + jnp.sum(h * wlb_ref[...], axis=1, keepdims=True)
                    + bl_ref[...])


def _tc_dcn(emb2d, Xv, E, P, uc, W1, b1, W2, b2, wlb, bl2,
            bm=256, interpret=False):
    nb = emb2d.shape[0]
    nblk = nb // bm
    full = lambda shape: pl.BlockSpec(shape, lambda i: (0, 0))
    out = pl.pallas_call(
        _tc_body,
        grid=(nblk,),
        in_specs=[
            pl.BlockSpec((bm, FD), lambda i: (i, 0)),
            pl.BlockSpec((bm, F_), lambda i: (i, 0)),
            full((F_, FD)),
            full((FD, 8)),
            full((1, 8)),
            full((FD, H1)),
            full((1, H1)),
            full((H1, H2)),
            full((1, H2)),
            full((1, H2)),
            full((1, 1)),
        ],
        out_specs=pl.BlockSpec((bm, 1), lambda i: (i, 0)),
        out_shape=jax.ShapeDtypeStruct((nb, 1), jnp.float32),
        compiler_params=pltpu.CompilerParams(
            dimension_semantics=("arbitrary",),
        ),
        interpret=interpret,
    )(emb2d, Xv, E, P, uc, W1, b1, W2, b2, wlb, bl2)
    return out[:, 0]


def kernel(Xi, Xv, tables, cross_w, cross_b, W1, b1, W2, b2, Wl, bl):
    tables_flat = tables.reshape(F_ * V, D)
    gidx = (Xi[:, :, 0].astype(jnp.int32)
            + (jnp.arange(F_, dtype=jnp.int32) * V)[None, :])

    E = jnp.kron(jnp.eye(F_, dtype=jnp.bfloat16),
                 jnp.ones((1, D), dtype=jnp.bfloat16))
    wlt = Wl[:FD, 0]
    wlb = Wl[FD:, 0][None, :]
    bl2 = bl.reshape(1, 1)
    P = jnp.stack([cross_w[0], cross_w[1], cross_b[0], cross_b[1], wlt,
                   jnp.zeros_like(wlt), jnp.zeros_like(wlt),
                   jnp.zeros_like(wlt)], axis=1)
    u = jnp.sum(cross_w * wlt[None, :], axis=1)
    c = jnp.sum(cross_b * wlt[None, :])
    uc = jnp.concatenate([u, c[None], jnp.zeros((4,), jnp.float32)]
                         ).reshape(1, 8)

    nsplit = 2
    bc = B // nsplit
    nch = bc * F_ // (_NW * _CH)
    outs = []
    for k in range(nsplit):
        gk = gidx[k * bc:(k + 1) * bc].reshape(_NW, nch, _CH)
        emb = _sc_gather(tables_flat, gk).reshape(bc, FD)
        outs.append(_tc_dcn(emb, Xv[k * bc:(k + 1) * bc], E, P, uc,
                            W1.astype(jnp.bfloat16), b1.reshape(1, H1),
                            W2.astype(jnp.bfloat16), b2.reshape(1, H2),
                            wlb, bl2))
    return jnp.concatenate(outs)


# speedup vs baseline: 1.0414x; 1.0414x over previous
import functools

import jax
import jax.numpy as jnp
from jax import lax
from jax.experimental import pallas as pl
from jax.experimental.pallas import tpu as pltpu
from jax.experimental.pallas import tpu_sc as plsc

B, F_, V, D = 4096, 26, 1000, 128
H1, H2 = 1024, 1024
CROSS_DEPTH = 3
FD = F_ * D

_NC, _NS = 2, 16
_NW = _NC * _NS
_CH = 128


def _gather_body(nch, tab, idx, out, idx_v, rows0, rows1, sem0, sem1):
    wid = lax.axis_index("s") * _NC + lax.axis_index("c")
    base = wid * nch
    pltpu.sync_copy(idx.at[wid], idx_v)

    def _start(ci, buf, sem):
        pltpu.async_copy(tab.at[idx_v.at[ci]], buf, sem)

    def _drain(ci, buf, sem):
        pltpu.make_async_copy(tab.at[idx_v.at[ci]], buf, sem).wait()
        off = pl.multiple_of((base + ci) * _CH, _CH)
        pltpu.sync_copy(buf, out.at[pl.ds(off, _CH)])

    bufs = ((rows0, sem0), (rows1, sem1))
    if nch <= 16:
        _start(0, rows0, sem0)
        if nch > 1:
            _start(1, rows1, sem1)
        for ci in range(nch):
            _drain(ci, *bufs[ci % 2])
            if ci + 2 < nch:
                _start(ci + 2, *bufs[ci % 2])
    else:
        assert nch % 2 == 0
        _start(0, rows0, sem0)
        _start(1, rows1, sem1)

        def step(i, carry):
            _drain(2 * i, rows0, sem0)
            _start(2 * i + 2, rows0, sem0)
            _drain(2 * i + 1, rows1, sem1)
            _start(2 * i + 3, rows1, sem1)
            return carry

        lax.fori_loop(0, nch // 2 - 1, step, 0)
        _drain(nch - 2, rows0, sem0)
        _drain(nch - 1, rows1, sem1)


def _sc_gather(tables_flat, gidx3d):
    nw, nch, ch = gidx3d.shape
    mesh = plsc.VectorSubcoreMesh(core_axis_name="c", subcore_axis_name="s")
    k = functools.partial(
        pl.kernel,
        mesh=mesh,
        out_type=jax.ShapeDtypeStruct((nw * nch * ch, D), jnp.float32),
        scratch_types=[
            pltpu.VMEM((nch, ch), jnp.int32),
            pltpu.VMEM((ch, D), jnp.float32),
            pltpu.VMEM((ch, D), jnp.float32),
            pltpu.SemaphoreType.DMA,
            pltpu.SemaphoreType.DMA,
        ],
    )(functools.partial(_gather_body, nch))
    return k(tables_flat, gidx3d)


def _tc_body(emb_ref, xv_ref, e_ref, p_ref, uc_ref, w1_ref, b1_ref,
             w2_ref, b2_ref, wlb_ref, bl_ref, out_ref):
    xv = xv_ref[...]
    xv_hi = xv.astype(jnp.bfloat16)
    xv_lo = (xv - xv_hi.astype(jnp.float32)).astype(jnp.bfloat16)
    e = e_ref[...]
    xv_wide = (jnp.dot(xv_hi, e, preferred_element_type=jnp.float32)
               + jnp.dot(xv_lo, e, preferred_element_type=jnp.float32))
    x0 = emb_ref[...] * xv_wide
    a = jnp.sum(x0 * x0, axis=1, keepdims=True)
    t = jnp.dot(x0, p_ref[...], preferred_element_type=jnp.float32)
    p0 = t[:, 0:1]
    p1 = t[:, 1:2]
    q0 = t[:, 2:3]
    q1 = t[:, 3:4]
    r0 = t[:, 4:5]
    s0 = a
    s1 = s0 * p0 + q0 + a
    s2 = s1 * p1 + q1 + s1
    uc = uc_ref[...]
    cross = (r0 + s0 * uc[:, 0:1] + s1 * uc[:, 1:2] + s2 * uc[:, 2:3]
             + uc[:, 3:4])
    h = jnp.dot(x0, w1_ref[...], preferred_element_type=jnp.float32)
    h = jnp.maximum(h + b1_ref[...], 0.0)
    h = jnp.dot(h, w2_ref[...], preferred_element_type=jnp.float32)
    h = jnp.maximum(h + b2_ref[...], 0.0)
    out_ref[...] = (cross
                    + jnp.sum(h * wlb_ref[...], axis=1, keepdims=True)
                    + bl_ref[...])


def _tc_dcn(emb2d, Xv, E, P, uc, W1, b1, W2, b2, wlb, bl2,
            bm=256, interpret=False):
    nb = emb2d.shape[0]
    nblk = nb // bm
    full = lambda shape: pl.BlockSpec(shape, lambda i: (0, 0))
    out = pl.pallas_call(
        _tc_body,
        grid=(nblk,),
        in_specs=[
            pl.BlockSpec((bm, FD), lambda i: (i, 0)),
            pl.BlockSpec((bm, F_), lambda i: (i, 0)),
            full((F_, FD)),
            full((FD, 8)),
            full((1, 8)),
            full((FD, H1)),
            full((1, H1)),
            full((H1, H2)),
            full((1, H2)),
            full((1, H2)),
            full((1, 1)),
        ],
        out_specs=pl.BlockSpec((bm, 1), lambda i: (i, 0)),
        out_shape=jax.ShapeDtypeStruct((nb, 1), jnp.float32),
        compiler_params=pltpu.CompilerParams(
            dimension_semantics=("arbitrary",),
        ),
        interpret=interpret,
    )(emb2d, Xv, E, P, uc, W1, b1, W2, b2, wlb, bl2)
    return out[:, 0]


def kernel(Xi, Xv, tables, cross_w, cross_b, W1, b1, W2, b2, Wl, bl):
    tables_flat = tables.reshape(F_ * V, D)
    gidx = (Xi[:, :, 0].astype(jnp.int32)
            + (jnp.arange(F_, dtype=jnp.int32) * V)[None, :])

    E = jnp.kron(jnp.eye(F_, dtype=jnp.bfloat16),
                 jnp.ones((1, D), dtype=jnp.bfloat16))
    wlt = Wl[:FD, 0]
    wlb = Wl[FD:, 0][None, :]
    bl2 = bl.reshape(1, 1)
    P = jnp.stack([cross_w[0], cross_w[1], cross_b[0], cross_b[1], wlt,
                   jnp.zeros_like(wlt), jnp.zeros_like(wlt),
                   jnp.zeros_like(wlt)], axis=1)
    u = jnp.sum(cross_w * wlt[None, :], axis=1)
    c = jnp.sum(cross_b * wlt[None, :])
    uc = jnp.concatenate([u, c[None], jnp.zeros((4,), jnp.float32)]
                         ).reshape(1, 8)

    nsplit = 1
    bc = B // nsplit
    nch = bc * F_ // (_NW * _CH)
    outs = []
    for k in range(nsplit):
        gk = gidx[k * bc:(k + 1) * bc].reshape(_NW, nch, _CH)
        emb = _sc_gather(tables_flat, gk).reshape(bc, FD)
        outs.append(_tc_dcn(emb, Xv[k * bc:(k + 1) * bc], E, P, uc,
                            W1, b1.reshape(1, H1),
                            W2, b2.reshape(1, H2),
                            wlb, bl2))
    return jnp.concatenate(outs)


# speedup vs baseline: 1.4128x; 1.3567x over previous
import functools

import jax
import jax.numpy as jnp
from jax import lax
from jax.experimental import pallas as pl
from jax.experimental.pallas import tpu as pltpu
from jax.experimental.pallas import tpu_sc as plsc

B, F_, V, D = 4096, 26, 1000, 128
H1, H2 = 1024, 1024
CROSS_DEPTH = 3
FD = F_ * D

_NC, _NS = 2, 16
_NW = _NC * _NS
_CH = 128


def _gather_body(nch, bw, tab, idx, out, idx_v, rows0, rows1, sem0, sem1):
    wid = lax.axis_index("s") * _NC + lax.axis_index("c")
    row0 = pl.multiple_of(wid * bw, 8)
    pltpu.sync_copy(idx.at[wid], idx_v)

    def _start(ci, buf, sem):
        pltpu.async_copy(tab.at[idx_v.at[ci]], buf, sem)

    def _drain(ci, buf, sem):
        pltpu.make_async_copy(tab.at[idx_v.at[ci]], buf, sem).wait()
        col = pl.multiple_of(ci * D, D)
        pltpu.sync_copy(buf, out.at[pl.ds(row0, bw), pl.ds(col, D)])

    bufs = ((rows0, sem0), (rows1, sem1))
    if nch <= 16:
        _start(0, rows0, sem0)
        if nch > 1:
            _start(1, rows1, sem1)
        for ci in range(nch):
            _drain(ci, *bufs[ci % 2])
            if ci + 2 < nch:
                _start(ci + 2, *bufs[ci % 2])
    else:
        assert nch % 2 == 0
        _start(0, rows0, sem0)
        _start(1, rows1, sem1)

        def step(i, carry):
            _drain(2 * i, rows0, sem0)
            _start(2 * i + 2, rows0, sem0)
            _drain(2 * i + 1, rows1, sem1)
            _start(2 * i + 3, rows1, sem1)
            return carry

        lax.fori_loop(0, nch // 2 - 1, step, 0)
        _drain(nch - 2, rows0, sem0)
        _drain(nch - 1, rows1, sem1)


def _sc_gather(tables_flat, gidx3d):
    nw, nch, bw = gidx3d.shape
    mesh = plsc.VectorSubcoreMesh(core_axis_name="c", subcore_axis_name="s")
    k = functools.partial(
        pl.kernel,
        mesh=mesh,
        out_type=jax.ShapeDtypeStruct((nw * bw, nch * D), jnp.float32),
        scratch_types=[
            pltpu.VMEM((nch, bw), jnp.int32),
            pltpu.VMEM((bw, D), jnp.float32),
            pltpu.VMEM((bw, D), jnp.float32),
            pltpu.SemaphoreType.DMA,
            pltpu.SemaphoreType.DMA,
        ],
    )(functools.partial(_gather_body, nch, bw))
    return k(tables_flat, gidx3d)


def _tc_body(emb_ref, xv_ref, e_ref, p_ref, uc_ref, w1_ref, b1_ref,
             w2_ref, b2_ref, wlb_ref, bl_ref, out_ref):
    xv = xv_ref[...]
    xv_hi = xv.astype(jnp.bfloat16)
    xv_lo = (xv - xv_hi.astype(jnp.float32)).astype(jnp.bfloat16)
    e = e_ref[...]
    xv_wide = (jnp.dot(xv_hi, e, preferred_element_type=jnp.float32)
               + jnp.dot(xv_lo, e, preferred_element_type=jnp.float32))
    x0 = emb_ref[...] * xv_wide
    a = jnp.sum(x0 * x0, axis=1, keepdims=True)
    t = jnp.dot(x0, p_ref[...], preferred_element_type=jnp.float32)
    p0 = t[:, 0:1]
    p1 = t[:, 1:2]
    q0 = t[:, 2:3]
    q1 = t[:, 3:4]
    r0 = t[:, 4:5]
    s0 = a
    s1 = s0 * p0 + q0 + a
    s2 = s1 * p1 + q1 + s1
    uc = uc_ref[...]
    cross = (r0 + s0 * uc[:, 0:1] + s1 * uc[:, 1:2] + s2 * uc[:, 2:3]
             + uc[:, 3:4])
    h = jnp.dot(x0, w1_ref[...], preferred_element_type=jnp.float32)
    h = jnp.maximum(h + b1_ref[...], 0.0)
    h = jnp.dot(h, w2_ref[...], preferred_element_type=jnp.float32)
    h = jnp.maximum(h + b2_ref[...], 0.0)
    out_ref[...] = (cross
                    + jnp.sum(h * wlb_ref[...], axis=1, keepdims=True)
                    + bl_ref[...])


def _tc_dcn(emb2d, Xv, E, P, uc, W1, b1, W2, b2, wlb, bl2,
            bm=256, interpret=False):
    nb = emb2d.shape[0]
    nblk = nb // bm
    full = lambda shape: pl.BlockSpec(shape, lambda i: (0, 0))
    out = pl.pallas_call(
        _tc_body,
        grid=(nblk,),
        in_specs=[
            pl.BlockSpec((bm, FD), lambda i: (i, 0)),
            pl.BlockSpec((bm, F_), lambda i: (i, 0)),
            full((F_, FD)),
            full((FD, 8)),
            full((1, 8)),
            full((FD, H1)),
            full((1, H1)),
            full((H1, H2)),
            full((1, H2)),
            full((1, H2)),
            full((1, 1)),
        ],
        out_specs=pl.BlockSpec((bm, 1), lambda i: (i, 0)),
        out_shape=jax.ShapeDtypeStruct((nb, 1), jnp.float32),
        compiler_params=pltpu.CompilerParams(
            dimension_semantics=("arbitrary",),
        ),
        interpret=interpret,
    )(emb2d, Xv, E, P, uc, W1, b1, W2, b2, wlb, bl2)
    return out[:, 0]


def kernel(Xi, Xv, tables, cross_w, cross_b, W1, b1, W2, b2, Wl, bl):
    tables_flat = tables.reshape(F_ * V, D)
    gidx = (Xi[:, :, 0].astype(jnp.int32)
            + (jnp.arange(F_, dtype=jnp.int32) * V)[None, :])

    E = jnp.kron(jnp.eye(F_, dtype=jnp.bfloat16),
                 jnp.ones((1, D), dtype=jnp.bfloat16))
    wlt = Wl[:FD, 0]
    wlb = Wl[FD:, 0][None, :]
    bl2 = bl.reshape(1, 1)
    P = jnp.stack([cross_w[0], cross_w[1], cross_b[0], cross_b[1], wlt,
                   jnp.zeros_like(wlt), jnp.zeros_like(wlt),
                   jnp.zeros_like(wlt)], axis=1)
    u = jnp.sum(cross_w * wlt[None, :], axis=1)
    c = jnp.sum(cross_b * wlt[None, :])
    uc = jnp.concatenate([u, c[None], jnp.zeros((4,), jnp.float32)]
                         ).reshape(1, 8)

    bw = B // _NW
    gidx3d = gidx.reshape(_NW, bw, F_).transpose(0, 2, 1)
    emb = _sc_gather(tables_flat, gidx3d)
    return _tc_dcn(emb, Xv, E, P, uc, W1, b1.reshape(1, H1),
                   W2, b2.reshape(1, H2), wlb, bl2)


# speedup vs baseline: 1.4285x; 1.0111x over previous
import functools

import jax
import jax.numpy as jnp
from jax import lax
from jax.experimental import pallas as pl
from jax.experimental.pallas import tpu as pltpu
from jax.experimental.pallas import tpu_sc as plsc

B, F_, V, D = 4096, 26, 1000, 128
H1, H2 = 1024, 1024
CROSS_DEPTH = 3
FD = F_ * D

_NC, _NS = 2, 16
_NW = _NC * _NS
_CH = 128


def _gather_body(nch, bw, tab, idx, out, idx_v, rows0, rows1, sem0, sem1):
    wid = lax.axis_index("s") * _NC + lax.axis_index("c")
    row0 = pl.multiple_of(wid * bw, 8)
    pltpu.sync_copy(idx.at[wid], idx_v)

    def _start(ci, buf, sem):
        pltpu.async_copy(tab.at[idx_v.at[ci]], buf, sem)

    def _drain(ci, buf, sem):
        pltpu.make_async_copy(tab.at[idx_v.at[ci]], buf, sem).wait()
        col = pl.multiple_of(ci * D, D)
        pltpu.sync_copy(buf, out.at[pl.ds(row0, bw), pl.ds(col, D)])

    bufs = ((rows0, sem0), (rows1, sem1))
    if nch <= 16:
        _start(0, rows0, sem0)
        if nch > 1:
            _start(1, rows1, sem1)
        for ci in range(nch):
            _drain(ci, *bufs[ci % 2])
            if ci + 2 < nch:
                _start(ci + 2, *bufs[ci % 2])
    else:
        assert nch % 2 == 0
        _start(0, rows0, sem0)
        _start(1, rows1, sem1)

        def step(i, carry):
            _drain(2 * i, rows0, sem0)
            _start(2 * i + 2, rows0, sem0)
            _drain(2 * i + 1, rows1, sem1)
            _start(2 * i + 3, rows1, sem1)
            return carry

        lax.fori_loop(0, nch // 2 - 1, step, 0)
        _drain(nch - 2, rows0, sem0)
        _drain(nch - 1, rows1, sem1)


def _sc_gather(tables_flat, gidx3d):
    nw, nch, bw = gidx3d.shape
    mesh = plsc.VectorSubcoreMesh(core_axis_name="c", subcore_axis_name="s")
    k = functools.partial(
        pl.kernel,
        mesh=mesh,
        out_type=jax.ShapeDtypeStruct((nw * bw, nch * D), jnp.float32),
        scratch_types=[
            pltpu.VMEM((nch, bw), jnp.int32),
            pltpu.VMEM((bw, D), jnp.float32),
            pltpu.VMEM((bw, D), jnp.float32),
            pltpu.SemaphoreType.DMA,
            pltpu.SemaphoreType.DMA,
        ],
    )(functools.partial(_gather_body, nch, bw))
    return k(tables_flat, gidx3d)


def _tc_body(emb_ref, xv_ref, e_ref, p_ref, uc_ref, w1_ref, b1_ref,
             w2_ref, b2_ref, wlb_ref, bl_ref, out_ref):
    xv = xv_ref[...]
    xv_hi = xv.astype(jnp.bfloat16)
    xv_lo = (xv - xv_hi.astype(jnp.float32)).astype(jnp.bfloat16)
    e = e_ref[...]
    xv_wide = (jnp.dot(xv_hi, e, preferred_element_type=jnp.float32)
               + jnp.dot(xv_lo, e, preferred_element_type=jnp.float32))
    x0 = emb_ref[...] * xv_wide
    a = jnp.sum(x0 * x0, axis=1, keepdims=True)
    t = jnp.dot(x0, p_ref[...], preferred_element_type=jnp.float32)
    p0 = t[:, 0:1]
    p1 = t[:, 1:2]
    q0 = t[:, 2:3]
    q1 = t[:, 3:4]
    r0 = t[:, 4:5]
    s0 = a
    s1 = s0 * p0 + q0 + a
    s2 = s1 * p1 + q1 + s1
    uc = uc_ref[...]
    cross = (r0 + s0 * uc[:, 0:1] + s1 * uc[:, 1:2] + s2 * uc[:, 2:3]
             + uc[:, 3:4])
    h = jnp.dot(x0, w1_ref[...], preferred_element_type=jnp.float32)
    h = jnp.maximum(h + b1_ref[...], 0.0)
    h = jnp.dot(h, w2_ref[...], preferred_element_type=jnp.float32)
    h = jnp.maximum(h + b2_ref[...], 0.0)
    out_ref[...] = (cross
                    + jnp.sum(h * wlb_ref[...], axis=1, keepdims=True)
                    + bl_ref[...])


def _tc_dcn(emb2d, Xv, E, P, uc, W1, b1, W2, b2, wlb, bl2,
            bm=512, interpret=False):
    nb = emb2d.shape[0]
    nblk = nb // bm
    full = lambda shape: pl.BlockSpec(shape, lambda i: (0, 0))
    out = pl.pallas_call(
        _tc_body,
        grid=(nblk,),
        in_specs=[
            pl.BlockSpec((bm, FD), lambda i: (i, 0)),
            pl.BlockSpec((bm, F_), lambda i: (i, 0)),
            full((F_, FD)),
            full((FD, 8)),
            full((1, 8)),
            full((FD, H1)),
            full((1, H1)),
            full((H1, H2)),
            full((1, H2)),
            full((1, H2)),
            full((1, 1)),
        ],
        out_specs=pl.BlockSpec((bm, 1), lambda i: (i, 0)),
        out_shape=jax.ShapeDtypeStruct((nb, 1), jnp.float32),
        compiler_params=pltpu.CompilerParams(
            dimension_semantics=("arbitrary",),
        ),
        interpret=interpret,
    )(emb2d, Xv, E, P, uc, W1, b1, W2, b2, wlb, bl2)
    return out[:, 0]


def kernel(Xi, Xv, tables, cross_w, cross_b, W1, b1, W2, b2, Wl, bl):
    tables_flat = tables.reshape(F_ * V, D)
    gidx = (Xi[:, :, 0].astype(jnp.int32)
            + (jnp.arange(F_, dtype=jnp.int32) * V)[None, :])

    E = jnp.kron(jnp.eye(F_, dtype=jnp.bfloat16),
                 jnp.ones((1, D), dtype=jnp.bfloat16))
    wlt = Wl[:FD, 0]
    wlb = Wl[FD:, 0][None, :]
    bl2 = bl.reshape(1, 1)
    P = jnp.stack([cross_w[0], cross_w[1], cross_b[0], cross_b[1], wlt,
                   jnp.zeros_like(wlt), jnp.zeros_like(wlt),
                   jnp.zeros_like(wlt)], axis=1)
    u = jnp.sum(cross_w * wlt[None, :], axis=1)
    c = jnp.sum(cross_b * wlt[None, :])
    uc = jnp.concatenate([u, c[None], jnp.zeros((4,), jnp.float32)]
                         ).reshape(1, 8)

    bw = B // _NW
    gidx3d = gidx.reshape(_NW, bw, F_).transpose(0, 2, 1)
    emb = _sc_gather(tables_flat, gidx3d)
    return _tc_dcn(emb, Xv, E, P, uc, W1, b1.reshape(1, H1),
                   W2, b2.reshape(1, H2), wlb, bl2)
